# baseline (speedup 1.0000x reference)
import functools

import jax
import jax.numpy as jnp
from jax import lax
from jax.experimental import pallas as pl
from jax.experimental.pallas import tpu as pltpu
from jax.experimental.pallas import tpu_sc as plsc

NC, NS, L = 2, 16, 16
NW = NC * NS
B, S, D = 4, 2048, 768
T = B * S
P = S // NW
NJ = D // L
JW = 6
NJB = NJ // JW
EPS = 1e-5

C = 32
NCHUNK = (B * P) // C


def _rsqrt(x):
    i = lax.bitcast_convert_type(x, jnp.int32)
    i = jnp.int32(0x5F3759DF) - lax.shift_right_logical(i, 1)
    y = lax.bitcast_convert_type(i, jnp.float32)
    for _ in range(3):
        y = y * (1.5 - 0.5 * x * y * y)
    return y


_GDN = lax.GatherDimensionNumbers(
    offset_dims=(), collapsed_slice_dims=(0,), start_index_map=(0,))


def _permute(v, perm):
    return lax.gather(v, perm[:, None], _GDN, slice_sizes=(1,),
                      mode=lax.GatherScatterMode.PROMISE_IN_BOUNDS)


def _lanesum(v):
    lane = lax.iota(jnp.int32, L)
    for sh in (1, 2, 4, 8):
        perm = lax.bitwise_and(lane + sh, L - 1)
        v = v + _permute(v, perm)
    return v


def _body(ids_hbm, tt_hbm, tok_hbm, pos_hbm, typ_hbm, g_hbm, bta_hbm, out_hbm,
          idxall, ttall, tokbufs, ppt2, tvbuf, gbuf, bbuf, mstat, rstat,
          sems, osems):
    wid = lax.axis_index("s") * NC + lax.axis_index("c")
    p0 = wid * P

    pltpu.sync_copy(typ_hbm, tvbuf)
    pltpu.sync_copy(g_hbm, gbuf)
    pltpu.sync_copy(bta_hbm, bbuf)
    for bi in range(B):
        pltpu.sync_copy(ids_hbm.at[pl.ds(bi * S + p0, P)],
                        idxall.at[pl.ds(bi * P, P)])
        pltpu.sync_copy(tt_hbm.at[pl.ds(bi * S + p0, P)],
                        ttall.at[pl.ds(bi * P, P)])

    def _offsets(ci):
        hb = lax.shift_right_logical(ci, 2) * C
        bi = lax.bitwise_and(ci, 3)
        return bi * P + hb, bi * S + p0 + hb, hb

    def _start_gather(ci, k):
        ioff, _, _ = _offsets(ci)
        pltpu.async_copy(tok_hbm.at[idxall.at[pl.ds(ioff, C)]],
                         tokbufs[k], sems[k])

    def _wait_gather(ci, k):
        ioff, _, _ = _offsets(ci)
        pltpu.make_async_copy(
            tok_hbm.at[idxall.at[pl.ds(ioff, C)]], tokbufs[k], sems[k]).wait()

    def _out_copy(ci, k):
        _, base, _ = _offsets(ci)
        return pltpu.make_async_copy(
            tokbufs[k], out_hbm.at[pl.ds(base, C)], osems[k])

    _start_gather(0, 0)

    @pl.loop(0, NCHUNK, step=2)
    def _chunk2(ci0):
        for k in range(2):
            ci = ci0 + k
            ioff, base, hb = _offsets(ci)

            @pl.when(lax.bitwise_and(ci, 3) == 0)
            def _():
                pltpu.sync_copy(pos_hbm.at[pl.ds(p0 + hb, C)], ppt2.at[0])
                pltpu.sync_copy(pos_hbm.at[pl.ds(p0 + hb, C)], ppt2.at[1])

                @plsc.parallel_loop(0, C)
                def _rb(r):
                    for j in range(NJ):
                        sl = pl.ds(j * L, L)
                        for q in range(2):
                            ppt2[q, r, sl] = ppt2[q, r, sl] + tvbuf[q, sl]

            nxt = ci + 1

            @pl.when(nxt < NCHUNK)
            def _():
                @pl.when(nxt >= 2)
                def _():
                    _out_copy(nxt - 2, 1 - k).wait()

                _start_gather(nxt, 1 - k)

            tokbuf = tokbufs[k]
            _wait_gather(ci, k)

            @plsc.parallel_loop(0, C)
            def _token(t):
                tt = ttall[pl.ds(ioff + t, L)][0]
                accs = [jnp.zeros((L,), jnp.float32) for _ in range(2)]
                acc2s = [jnp.zeros((L,), jnp.float32) for _ in range(2)]
                for j in range(NJ):
                    sl = pl.ds(j * L, L)
                    x = tokbuf[t, sl] + ppt2[tt, t, sl]
                    tokbuf[t, sl] = x
                    accs[j % 2] = accs[j % 2] + x
                    acc2s[j % 2] = acc2s[j % 2] + x * x
                mb = _lanesum(accs[0] + accs[1]) * (1.0 / D)
                rb = _rsqrt(
                    _lanesum(acc2s[0] + acc2s[1]) * (1.0 / D) - mb * mb + EPS)
                mstat[t, :] = mb
                rstat[t, :] = rb

            for jb in range(NJB):
                gs = [gbuf[pl.ds((jb * JW + jj) * L, L)] for jj in range(JW)]
                bs = [bbuf[pl.ds((jb * JW + jj) * L, L)] for jj in range(JW)]

                @plsc.parallel_loop(0, C)
                def _norm(t):
                    mbv = mstat[t, :]
                    rbv = rstat[t, :]
                    for jj in range(JW):
                        sl = pl.ds((jb * JW + jj) * L, L)
                        tokbuf[t, sl] = (
                            (tokbuf[t, sl] - mbv) * rbv * gs[jj] + bs[jj])

            _out_copy(ci, k).start()

    _out_copy(NCHUNK - 2, 0).wait()
    _out_copy(NCHUNK - 1, 1).wait()


@functools.cache
def _sc_embed_fn():
    return functools.partial(
        pl.kernel,
        out_type=jax.ShapeDtypeStruct((T, D), jnp.float32),
        mesh=plsc.VectorSubcoreMesh(
            core_axis_name="c", subcore_axis_name="s",
            num_cores=NC, num_subcores=NS,
        ),
        scratch_types=[
            pltpu.VMEM((B * P,), jnp.int32),
            pltpu.VMEM((B * P + L,), jnp.int32),
            [pltpu.VMEM((C, D), jnp.float32)] * 2,
            pltpu.VMEM((2, C, D), jnp.float32),
            pltpu.VMEM((2, D), jnp.float32),
            pltpu.VMEM((D,), jnp.float32),
            pltpu.VMEM((D,), jnp.float32),
            pltpu.VMEM((C, L), jnp.float32),
            pltpu.VMEM((C, L), jnp.float32),
            [pltpu.SemaphoreType.DMA] * 2,
            [pltpu.SemaphoreType.DMA] * 2,
        ],
    )(_body)


def kernel(input_ids, token_type_ids, token_table, position_table, type_table,
           ln_gamma, ln_beta):
    ids = input_ids.reshape(-1).astype(jnp.int32)
    tts = token_type_ids.reshape(-1).astype(jnp.int32)
    out = _sc_embed_fn()(ids, tts, token_table, position_table, type_table,
                         ln_gamma, ln_beta)
    return out.reshape(B, S, D)
